# baseline (speedup 1.0000x reference)
import functools

import jax
import jax.numpy as jnp
from jax import lax
from jax.experimental import pallas as pl
from jax.experimental.pallas import tpu as pltpu
from jax.experimental.pallas import tpu_sc as plsc

_LEAKY = 0.01
_EPS = 1e-5
_BINS = 512
_GROUP = 128
_RB = 512
_CHUNK = 128


def _logits(x_ref, nb):
    x_bf = x_ref[0].astype(jnp.bfloat16)
    xb = x_ref[0, :, pl.ds(nb * _RB, _RB)].astype(jnp.bfloat16)
    return lax.dot_general(
        xb, x_bf, dimension_numbers=(((0,), (0,)), ((), ())),
        preferred_element_type=jnp.float32)


def _quant_block(x_ref, qs_ref):
    nb = pl.program_id(1)
    N = x_ref.shape[2]
    logits = _logits(x_ref, nb)
    row_max = jnp.max(logits, axis=1, keepdims=True)
    row_min = jnp.min(logits, axis=1, keepdims=True)
    rng = row_max - row_min
    scale = jnp.where(rng > 0, _BINS / rng, 0.0)
    q = jnp.minimum(jnp.floor((logits - row_min) * scale), _BINS - 1)
    for g in range(_RB // _GROUP):
        qs_ref[pl.ds(g * N, N), :] = q[g * _GROUP:(g + 1) * _GROUP, :].T


def _sc_hist(qs_hbm, hist_hbm, buf0, buf1, hist, sem0, sem1):
    w = lax.axis_index("s") * 2 + lax.axis_index("c")
    n_groups = qs_hbm.shape[0] * 128 // (4096 * _GROUP)
    ngw = n_groups // 32
    qs_rows_per_group = 4096 * _GROUP // 128
    nchunk = qs_rows_per_group // _CHUNK
    bufs = (buf0, buf1)
    sems = (sem0, sem1)
    ones = jnp.ones((16,), jnp.float32)
    consts = [lax.iota(jnp.int32, 16) * _BINS + l * 16 * _BINS
              for l in range(8)]

    for gi in range(ngw):
        g = w * ngw + gi

        def clr(i, _):
            hist[pl.ds(i * 16, 16)] = jnp.zeros((16,), jnp.float32)
            return 0
        lax.fori_loop(0, _GROUP * _BINS // 16, clr, 0)

        base = pl.multiple_of(g * qs_rows_per_group, _CHUNK)
        cp = pltpu.async_copy(qs_hbm.at[pl.ds(base, _CHUNK)], bufs[0],
                              sems[0])
        for ch in range(nchunk):
            cp.wait()
            if ch + 1 < nchunk:
                nxt = pl.multiple_of(base + (ch + 1) * _CHUNK, _CHUNK)
                cp = pltpu.async_copy(
                    qs_hbm.at[pl.ds(nxt, _CHUNK)],
                    bufs[(ch + 1) % 2], sems[(ch + 1) % 2])
            cur = bufs[ch % 2]

            def body(i, _):
                for l in range(8):
                    v = cur[i, pl.ds(l * 16, 16)]
                    idx = v.astype(jnp.int32) + consts[l]
                    plsc.addupdate_scatter(hist, [idx], ones)
                return 0
            lax.fori_loop(0, _CHUNK, body, 0)

        hb = pl.multiple_of(g * (_GROUP * _BINS), _GROUP * _BINS)
        pltpu.sync_copy(hist, hist_hbm.at[pl.ds(hb, _GROUP * _BINS)])


def _fused_block(x_ref, w_ref, h_ref, y_ref, sum_ref, ssq_ref, s_ref):
    nb = pl.program_id(1)
    N = x_ref.shape[2]
    k = int(round(N * 0.9))
    n_drop = N - k

    x_bf = x_ref[0].astype(jnp.bfloat16)

    @pl.when(nb == 0)
    def _():
        s_ref[...] = lax.dot_general(
            x_bf, w_ref[...].astype(jnp.bfloat16),
            dimension_numbers=(((0,), (0,)), ((), ())),
            preferred_element_type=jnp.float32,
        ).astype(jnp.bfloat16)

    logits = _logits(x_ref, nb)
    row_max = jnp.max(logits, axis=1, keepdims=True)
    row_min = jnp.min(logits, axis=1, keepdims=True)
    rng = row_max - row_min
    e = jnp.exp(logits - row_max)
    denom = jnp.sum(e, axis=1, keepdims=True)

    h = h_ref[...]
    c = h
    for sh in (1, 2, 4, 8, 16, 32, 64):
        z = jnp.zeros((c.shape[0], sh), jnp.float32)
        c = c + jnp.concatenate([z, c[:, :-sh]], axis=1)
    t = c[:, 127:128]
    rows4 = c.shape[0]
    pos = lax.broadcasted_iota(jnp.int32, (rows4, 1), 0) % 4
    offs = jnp.zeros((rows4, 1), jnp.float32)
    for sh in (1, 2, 3):
        z = jnp.zeros((sh, 1), jnp.float32)
        shifted = jnp.concatenate([z, t[:-sh, :]], axis=0)
        offs = offs + jnp.where(pos >= sh, shifted, 0.0)
    cum = c + offs
    ind = jnp.where(cum <= float(n_drop), 1.0, 0.0)
    s = jnp.sum(ind, axis=1, keepdims=True)
    bstarp1 = jnp.sum(s.reshape(-1, 4), axis=1, keepdims=True)
    thr = row_min + bstarp1 * (rng / _BINS)

    p = jnp.where(logits >= thr, e, 0.0) * (1.0 / denom)
    out = lax.dot_general(
        p.astype(jnp.bfloat16), s_ref[...],
        dimension_numbers=(((1,), (0,)), ((), ())),
        preferred_element_type=jnp.float32)
    z = jnp.where(out >= 0.0, out, _LEAKY * out)
    y_ref[0] = z
    sum_ref[0, 0] = jnp.sum(z, axis=0)
    ssq_ref[0, 0] = jnp.sum(z * z, axis=0)


def _bn_block(sum_ref, ssq_ref, g_ref, b_ref, y_ref, out_ref, *, count):
    tot = jnp.sum(sum_ref[:, 0, :], axis=0, keepdims=True)
    tot2 = jnp.sum(ssq_ref[:, 0, :], axis=0, keepdims=True)
    mean = tot / count
    var = tot2 / count - mean * mean
    inv = lax.rsqrt(var + _EPS)
    scale = inv * g_ref[...]
    shift = b_ref[...] - mean * scale
    z = y_ref[0] * scale + shift
    out_ref[0] = z.T


def kernel(input, W, gamma, beta):
    B, C, N = input.shape
    O = W.shape[1]
    NB = N // _RB

    qs = pl.pallas_call(
        _quant_block,
        grid=(B, NB),
        in_specs=[pl.BlockSpec((1, C, N), lambda b, n: (b, 0, 0))],
        out_specs=pl.BlockSpec((_RB * N // 128, 128),
                               lambda b, n: (b * NB + n, 0)),
        out_shape=jax.ShapeDtypeStruct((B * N * N // 128, 128), jnp.float32),
    )(input)

    mesh = plsc.VectorSubcoreMesh(core_axis_name="c", subcore_axis_name="s")
    hist = pl.kernel(
        _sc_hist,
        mesh=mesh,
        compiler_params=pltpu.CompilerParams(needs_layout_passes=False),
        out_type=jax.ShapeDtypeStruct((B * N * _BINS,), jnp.float32),
        scratch_types=[
            pltpu.VMEM((_CHUNK, 128), jnp.float32),
            pltpu.VMEM((_CHUNK, 128), jnp.float32),
            pltpu.VMEM((_GROUP * _BINS,), jnp.float32),
            pltpu.SemaphoreType.DMA,
            pltpu.SemaphoreType.DMA,
        ],
    )(qs)
    hist = hist.reshape(B * N * _BINS // 128, 128)

    y, s1, s2 = pl.pallas_call(
        _fused_block,
        grid=(B, NB),
        in_specs=[
            pl.BlockSpec((1, C, N), lambda b, n: (b, 0, 0)),
            pl.BlockSpec((C, O), lambda b, n: (0, 0)),
            pl.BlockSpec((_RB * _BINS // 128, 128),
                         lambda b, n: (b * NB + n, 0)),
        ],
        out_specs=[
            pl.BlockSpec((1, _RB, O), lambda b, n: (b, n, 0)),
            pl.BlockSpec((1, 1, O), lambda b, n: (b * NB + n, 0, 0)),
            pl.BlockSpec((1, 1, O), lambda b, n: (b * NB + n, 0, 0)),
        ],
        out_shape=[
            jax.ShapeDtypeStruct((B, N, O), jnp.float32),
            jax.ShapeDtypeStruct((B * NB, 1, O), jnp.float32),
            jax.ShapeDtypeStruct((B * NB, 1, O), jnp.float32),
        ],
        scratch_shapes=[pltpu.VMEM((N, O), jnp.bfloat16)],
    )(input, W, hist)

    out = pl.pallas_call(
        functools.partial(_bn_block, count=B * N),
        grid=(B, NB),
        in_specs=[
            pl.BlockSpec((B * NB, 1, O), lambda b, n: (0, 0, 0)),
            pl.BlockSpec((B * NB, 1, O), lambda b, n: (0, 0, 0)),
            pl.BlockSpec((1, O), lambda b, n: (0, 0)),
            pl.BlockSpec((1, O), lambda b, n: (0, 0)),
            pl.BlockSpec((1, _RB, O), lambda b, n: (b, n, 0)),
        ],
        out_specs=pl.BlockSpec((1, O, _RB), lambda b, n: (b, 0, n)),
        out_shape=jax.ShapeDtypeStruct((B, O, N), jnp.float32),
    )(s1, s2, gamma.reshape(1, O), beta.reshape(1, O), y)
    return out


# speedup vs baseline: 2.2199x; 2.2199x over previous
import functools

import jax
import jax.numpy as jnp
from jax import lax
from jax.experimental import pallas as pl
from jax.experimental.pallas import tpu as pltpu
from jax.experimental.pallas import tpu_sc as plsc

_LEAKY = 0.01
_EPS = 1e-5
_BINS = 512
_GROUP = 128
_RB = 512
_CHUNK = 128


def _logits(x_ref, nb):
    x_bf = x_ref[0].astype(jnp.bfloat16)
    xb = x_ref[0, :, pl.ds(nb * _RB, _RB)].astype(jnp.bfloat16)
    return lax.dot_general(
        xb, x_bf, dimension_numbers=(((0,), (0,)), ((), ())),
        preferred_element_type=jnp.float32)


def _quant_block(x_ref, qs_ref):
    nb = pl.program_id(1)
    N = x_ref.shape[2]
    logits = _logits(x_ref, nb)
    row_max = jnp.max(logits, axis=1, keepdims=True)
    row_min = jnp.min(logits, axis=1, keepdims=True)
    rng = row_max - row_min
    scale = jnp.where(rng > 0, _BINS / rng, 0.0)
    q = jnp.minimum(jnp.floor((logits - row_min) * scale), _BINS - 1)
    for g in range(_RB // _GROUP):
        qs_ref[pl.ds(g * N, N), :] = q[g * _GROUP:(g + 1) * _GROUP, :].T


def _sc_hist(qs_hbm, hist_hbm, buf0, buf1, hist, sem0, sem1):
    w = lax.axis_index("s") * 2 + lax.axis_index("c")
    n_groups = qs_hbm.shape[0] * 128 // (4096 * _GROUP)
    ngw = n_groups // 32
    qs_rows_per_group = 4096 * _GROUP // 128
    nchunk = qs_rows_per_group // _CHUNK
    bufs = (buf0, buf1)
    sems = (sem0, sem1)
    ones = jnp.ones((16,), jnp.float32)
    consts = [lax.iota(jnp.int32, 16) * _BINS + l * 16 * _BINS
              for l in range(8)]

    def start(buf, sem, base_row):
        pltpu.async_copy(
            qs_hbm.at[pl.ds(pl.multiple_of(base_row, _CHUNK), _CHUNK)],
            buf, sem)

    def wait(buf, sem):
        pltpu.make_async_copy(qs_hbm.at[pl.ds(0, _CHUNK)], buf, sem).wait()

    def process(buf):
        @plsc.parallel_loop(0, _CHUNK, unroll=2)
        def _(i):
            for l in range(8):
                v = buf[i, pl.ds(l * 16, 16)]
                idx = v.astype(jnp.int32) + consts[l]
                plsc.addupdate_scatter(hist, [idx], ones)

    for gi in range(ngw):
        g = w * ngw + gi

        @plsc.parallel_loop(0, _GROUP * _BINS // 16, unroll=4)
        def _(i):
            hist[pl.ds(i * 16, 16)] = jnp.zeros((16,), jnp.float32)

        base = g * qs_rows_per_group
        start(bufs[0], sems[0], base)

        def pair(j, _):
            c0 = base + 2 * j * _CHUNK
            wait(bufs[0], sems[0])
            start(bufs[1], sems[1], c0 + _CHUNK)
            process(bufs[0])
            wait(bufs[1], sems[1])

            @pl.when(j < nchunk // 2 - 1)
            def _():
                start(bufs[0], sems[0], c0 + 2 * _CHUNK)

            process(bufs[1])
            return 0
        lax.fori_loop(0, nchunk // 2, pair, 0)

        hb = pl.multiple_of(g * (_GROUP * _BINS), _GROUP * _BINS)
        pltpu.sync_copy(hist, hist_hbm.at[pl.ds(hb, _GROUP * _BINS)])


def _fused_block(x_ref, w_ref, h_ref, y_ref, sum_ref, ssq_ref, s_ref):
    nb = pl.program_id(1)
    N = x_ref.shape[2]
    k = int(round(N * 0.9))
    n_drop = N - k

    x_bf = x_ref[0].astype(jnp.bfloat16)

    @pl.when(nb == 0)
    def _():
        s_ref[...] = lax.dot_general(
            x_bf, w_ref[...].astype(jnp.bfloat16),
            dimension_numbers=(((0,), (0,)), ((), ())),
            preferred_element_type=jnp.float32,
        ).astype(jnp.bfloat16)

    logits = _logits(x_ref, nb)
    row_max = jnp.max(logits, axis=1, keepdims=True)
    row_min = jnp.min(logits, axis=1, keepdims=True)
    rng = row_max - row_min
    e = jnp.exp(logits - row_max)
    denom = jnp.sum(e, axis=1, keepdims=True)

    h = h_ref[...]
    c = h
    for sh in (1, 2, 4, 8, 16, 32, 64):
        z = jnp.zeros((c.shape[0], sh), jnp.float32)
        c = c + jnp.concatenate([z, c[:, :-sh]], axis=1)
    t = c[:, 127:128]
    rows4 = c.shape[0]
    pos = lax.broadcasted_iota(jnp.int32, (rows4, 1), 0) % 4
    offs = jnp.zeros((rows4, 1), jnp.float32)
    for sh in (1, 2, 3):
        z = jnp.zeros((sh, 1), jnp.float32)
        shifted = jnp.concatenate([z, t[:-sh, :]], axis=0)
        offs = offs + jnp.where(pos >= sh, shifted, 0.0)
    cum = c + offs
    ind = jnp.where(cum <= float(n_drop), 1.0, 0.0)
    s = jnp.sum(ind, axis=1, keepdims=True)
    bstarp1 = jnp.sum(s.reshape(-1, 4), axis=1, keepdims=True)
    thr = row_min + bstarp1 * (rng / _BINS)

    p = jnp.where(logits >= thr, e, 0.0) * (1.0 / denom)
    out = lax.dot_general(
        p.astype(jnp.bfloat16), s_ref[...],
        dimension_numbers=(((1,), (0,)), ((), ())),
        preferred_element_type=jnp.float32)
    z = jnp.where(out >= 0.0, out, _LEAKY * out)
    y_ref[0] = z
    sum_ref[0, 0] = jnp.sum(z, axis=0)
    ssq_ref[0, 0] = jnp.sum(z * z, axis=0)


def _bn_block(sum_ref, ssq_ref, g_ref, b_ref, y_ref, out_ref, *, count):
    tot = jnp.sum(sum_ref[:, 0, :], axis=0, keepdims=True)
    tot2 = jnp.sum(ssq_ref[:, 0, :], axis=0, keepdims=True)
    mean = tot / count
    var = tot2 / count - mean * mean
    inv = lax.rsqrt(var + _EPS)
    scale = inv * g_ref[...]
    shift = b_ref[...] - mean * scale
    z = y_ref[0] * scale + shift
    out_ref[0] = z.T


def kernel(input, W, gamma, beta):
    B, C, N = input.shape
    O = W.shape[1]
    NB = N // _RB

    qs = pl.pallas_call(
        _quant_block,
        grid=(B, NB),
        in_specs=[pl.BlockSpec((1, C, N), lambda b, n: (b, 0, 0))],
        out_specs=pl.BlockSpec((_RB * N // 128, 128),
                               lambda b, n: (b * NB + n, 0)),
        out_shape=jax.ShapeDtypeStruct((B * N * N // 128, 128), jnp.float32),
    )(input)

    mesh = plsc.VectorSubcoreMesh(core_axis_name="c", subcore_axis_name="s")
    hist = pl.kernel(
        _sc_hist,
        mesh=mesh,
        compiler_params=pltpu.CompilerParams(needs_layout_passes=False),
        out_type=jax.ShapeDtypeStruct((B * N * _BINS,), jnp.float32),
        scratch_types=[
            pltpu.VMEM((_CHUNK, 128), jnp.float32),
            pltpu.VMEM((_CHUNK, 128), jnp.float32),
            pltpu.VMEM((_GROUP * _BINS,), jnp.float32),
            pltpu.SemaphoreType.DMA,
            pltpu.SemaphoreType.DMA,
        ],
    )(qs)
    hist = hist.reshape(B * N * _BINS // 128, 128)

    y, s1, s2 = pl.pallas_call(
        _fused_block,
        grid=(B, NB),
        in_specs=[
            pl.BlockSpec((1, C, N), lambda b, n: (b, 0, 0)),
            pl.BlockSpec((C, O), lambda b, n: (0, 0)),
            pl.BlockSpec((_RB * _BINS // 128, 128),
                         lambda b, n: (b * NB + n, 0)),
        ],
        out_specs=[
            pl.BlockSpec((1, _RB, O), lambda b, n: (b, n, 0)),
            pl.BlockSpec((1, 1, O), lambda b, n: (b * NB + n, 0, 0)),
            pl.BlockSpec((1, 1, O), lambda b, n: (b * NB + n, 0, 0)),
        ],
        out_shape=[
            jax.ShapeDtypeStruct((B, N, O), jnp.float32),
            jax.ShapeDtypeStruct((B * NB, 1, O), jnp.float32),
            jax.ShapeDtypeStruct((B * NB, 1, O), jnp.float32),
        ],
        scratch_shapes=[pltpu.VMEM((N, O), jnp.bfloat16)],
    )(input, W, hist)

    out = pl.pallas_call(
        functools.partial(_bn_block, count=B * N),
        grid=(B, NB),
        in_specs=[
            pl.BlockSpec((B * NB, 1, O), lambda b, n: (0, 0, 0)),
            pl.BlockSpec((B * NB, 1, O), lambda b, n: (0, 0, 0)),
            pl.BlockSpec((1, O), lambda b, n: (0, 0)),
            pl.BlockSpec((1, O), lambda b, n: (0, 0)),
            pl.BlockSpec((1, _RB, O), lambda b, n: (b, n, 0)),
        ],
        out_specs=pl.BlockSpec((1, O, _RB), lambda b, n: (b, 0, n)),
        out_shape=jax.ShapeDtypeStruct((B, O, N), jnp.float32),
    )(s1, s2, gamma.reshape(1, O), beta.reshape(1, O), y)
    return out


# speedup vs baseline: 2.7391x; 1.2339x over previous
import functools

import jax
import jax.numpy as jnp
from jax import lax
from jax.experimental import pallas as pl
from jax.experimental.pallas import tpu as pltpu
from jax.experimental.pallas import tpu_sc as plsc

_LEAKY = 0.01
_EPS = 1e-5
_BINS = 256
_GROUP = 128
_RB = 512
_CHUNK = 256
_CPR = _BINS // 128


def _logits(x_ref, nb):
    x_bf = x_ref[0].astype(jnp.bfloat16)
    xb = x_ref[0, :, pl.ds(nb * _RB, _RB)].astype(jnp.bfloat16)
    return lax.dot_general(
        xb, x_bf, dimension_numbers=(((0,), (0,)), ((), ())),
        preferred_element_type=jnp.float32)


def _quant_block(x_ref, qs_ref):
    nb = pl.program_id(1)
    N = x_ref.shape[2]
    logits = _logits(x_ref, nb)
    row_max = jnp.max(logits, axis=1, keepdims=True)
    row_min = jnp.min(logits, axis=1, keepdims=True)
    rng = row_max - row_min
    scale = jnp.where(rng > 0, _BINS / rng, 0.0)
    q = jnp.minimum(jnp.floor((logits - row_min) * scale), _BINS - 1)
    qp = q[:, :N // 2] * float(_BINS) + q[:, N // 2:]
    for g in range(_RB // _GROUP):
        qs_ref[pl.ds(g * (N // 2), N // 2), :] = (
            qp[g * _GROUP:(g + 1) * _GROUP, :].T)


def _sc_hist(qs_hbm, hist_hbm, buf0, buf1, hist, sem0, sem1):
    w = lax.axis_index("s") * 2 + lax.axis_index("c")
    n_groups = qs_hbm.shape[0] * 128 * 2 // (4096 * _GROUP)
    ngw = n_groups // 32
    qs_rows_per_group = 4096 * _GROUP // 128 // 2
    nchunk = qs_rows_per_group // _CHUNK
    bufs = (buf0, buf1)
    sems = (sem0, sem1)
    ones = jnp.ones((16,), jnp.float32)
    consts = [lax.iota(jnp.int32, 16) * _BINS + l * 16 * _BINS
              for l in range(8)]

    def start(buf, sem, base_row):
        pltpu.async_copy(
            qs_hbm.at[pl.ds(pl.multiple_of(base_row, _CHUNK), _CHUNK)],
            buf, sem)

    def wait(buf, sem):
        pltpu.make_async_copy(qs_hbm.at[pl.ds(0, _CHUNK)], buf, sem).wait()

    def process(buf):
        @plsc.parallel_loop(0, _CHUNK, unroll=2)
        def _(i):
            for l in range(8):
                v = buf[i, pl.ds(l * 16, 16)]
                vi = v.astype(jnp.int32)
                hi = lax.shift_right_logical(vi, _BINS.bit_length() - 1)
                lo = lax.bitwise_and(vi, _BINS - 1)
                plsc.addupdate_scatter(hist, [hi + consts[l]], ones)
                plsc.addupdate_scatter(hist, [lo + consts[l]], ones)

    for gi in range(ngw):
        g = w * ngw + gi

        @plsc.parallel_loop(0, _GROUP * _BINS // 16, unroll=4)
        def _(i):
            hist[pl.ds(i * 16, 16)] = jnp.zeros((16,), jnp.float32)

        base = g * qs_rows_per_group
        start(bufs[0], sems[0], base)

        def pair(j, _):
            c0 = base + 2 * j * _CHUNK
            wait(bufs[0], sems[0])
            start(bufs[1], sems[1], c0 + _CHUNK)
            process(bufs[0])
            wait(bufs[1], sems[1])

            @pl.when(j < nchunk // 2 - 1)
            def _():
                start(bufs[0], sems[0], c0 + 2 * _CHUNK)

            process(bufs[1])
            return 0
        lax.fori_loop(0, nchunk // 2, pair, 0)

        hb = pl.multiple_of(g * (_GROUP * _BINS), _GROUP * _BINS)
        pltpu.sync_copy(hist, hist_hbm.at[pl.ds(hb, _GROUP * _BINS)])


def _fused_block(x_ref, w_ref, h_ref, y_ref, sum_ref, ssq_ref, s_ref):
    nb = pl.program_id(1)
    N = x_ref.shape[2]
    k = int(round(N * 0.9))
    n_drop = N - k

    x_bf = x_ref[0].astype(jnp.bfloat16)

    @pl.when(nb == 0)
    def _():
        s_ref[...] = lax.dot_general(
            x_bf, w_ref[...].astype(jnp.bfloat16),
            dimension_numbers=(((0,), (0,)), ((), ())),
            preferred_element_type=jnp.float32,
        ).astype(jnp.bfloat16)

    logits = _logits(x_ref, nb)
    row_max = jnp.max(logits, axis=1, keepdims=True)
    row_min = jnp.min(logits, axis=1, keepdims=True)
    rng = row_max - row_min
    e = jnp.exp(logits - row_max)
    denom = jnp.sum(e, axis=1, keepdims=True)

    h = h_ref[...]
    c = h
    for sh in (1, 2, 4, 8, 16, 32, 64):
        z = jnp.zeros((c.shape[0], sh), jnp.float32)
        c = c + jnp.concatenate([z, c[:, :-sh]], axis=1)
    t = c[:, 127:128]
    rowsc = c.shape[0]
    pos = lax.broadcasted_iota(jnp.int32, (rowsc, 1), 0) % _CPR
    offs = jnp.zeros((rowsc, 1), jnp.float32)
    for sh in range(1, _CPR):
        z = jnp.zeros((sh, 1), jnp.float32)
        shifted = jnp.concatenate([z, t[:-sh, :]], axis=0)
        offs = offs + jnp.where(pos >= sh, shifted, 0.0)
    cum = c + offs
    ind = jnp.where(cum <= float(n_drop), 1.0, 0.0)
    s = jnp.sum(ind, axis=1, keepdims=True)
    bstarp1 = jnp.sum(s.reshape(-1, _CPR), axis=1, keepdims=True)
    thr = row_min + bstarp1 * (rng / _BINS)

    p = jnp.where(logits >= thr, e, 0.0) * (1.0 / denom)
    out = lax.dot_general(
        p.astype(jnp.bfloat16), s_ref[...],
        dimension_numbers=(((1,), (0,)), ((), ())),
        preferred_element_type=jnp.float32)
    z = jnp.where(out >= 0.0, out, _LEAKY * out)
    y_ref[0] = z
    sum_ref[0, 0] = jnp.sum(z, axis=0)
    ssq_ref[0, 0] = jnp.sum(z * z, axis=0)


def _bn_block(sum_ref, ssq_ref, g_ref, b_ref, y_ref, out_ref, *, count):
    tot = jnp.sum(sum_ref[:, 0, :], axis=0, keepdims=True)
    tot2 = jnp.sum(ssq_ref[:, 0, :], axis=0, keepdims=True)
    mean = tot / count
    var = tot2 / count - mean * mean
    inv = lax.rsqrt(var + _EPS)
    scale = inv * g_ref[...]
    shift = b_ref[...] - mean * scale
    z = y_ref[0] * scale + shift
    out_ref[0] = z.T


def kernel(input, W, gamma, beta):
    B, C, N = input.shape
    O = W.shape[1]
    NB = N // _RB

    qs = pl.pallas_call(
        _quant_block,
        grid=(B, NB),
        in_specs=[pl.BlockSpec((1, C, N), lambda b, n: (b, 0, 0))],
        out_specs=pl.BlockSpec((_RB * N // 256, 128),
                               lambda b, n: (b * NB + n, 0)),
        out_shape=jax.ShapeDtypeStruct((B * N * N // 256, 128), jnp.float32),
    )(input)

    mesh = plsc.VectorSubcoreMesh(core_axis_name="c", subcore_axis_name="s")
    hist = pl.kernel(
        _sc_hist,
        mesh=mesh,
        compiler_params=pltpu.CompilerParams(needs_layout_passes=False),
        out_type=jax.ShapeDtypeStruct((B * N * _BINS,), jnp.float32),
        scratch_types=[
            pltpu.VMEM((_CHUNK, 128), jnp.float32),
            pltpu.VMEM((_CHUNK, 128), jnp.float32),
            pltpu.VMEM((_GROUP * _BINS,), jnp.float32),
            pltpu.SemaphoreType.DMA,
            pltpu.SemaphoreType.DMA,
        ],
    )(qs)
    hist = hist.reshape(B * N * _BINS // 128, 128)

    y, s1, s2 = pl.pallas_call(
        _fused_block,
        grid=(B, NB),
        in_specs=[
            pl.BlockSpec((1, C, N), lambda b, n: (b, 0, 0)),
            pl.BlockSpec((C, O), lambda b, n: (0, 0)),
            pl.BlockSpec((_RB * _BINS // 128, 128),
                         lambda b, n: (b * NB + n, 0)),
        ],
        out_specs=[
            pl.BlockSpec((1, _RB, O), lambda b, n: (b, n, 0)),
            pl.BlockSpec((1, 1, O), lambda b, n: (b * NB + n, 0, 0)),
            pl.BlockSpec((1, 1, O), lambda b, n: (b * NB + n, 0, 0)),
        ],
        out_shape=[
            jax.ShapeDtypeStruct((B, N, O), jnp.float32),
            jax.ShapeDtypeStruct((B * NB, 1, O), jnp.float32),
            jax.ShapeDtypeStruct((B * NB, 1, O), jnp.float32),
        ],
        scratch_shapes=[pltpu.VMEM((N, O), jnp.bfloat16)],
    )(input, W, hist)

    out = pl.pallas_call(
        functools.partial(_bn_block, count=B * N),
        grid=(B, NB),
        in_specs=[
            pl.BlockSpec((B * NB, 1, O), lambda b, n: (0, 0, 0)),
            pl.BlockSpec((B * NB, 1, O), lambda b, n: (0, 0, 0)),
            pl.BlockSpec((1, O), lambda b, n: (0, 0)),
            pl.BlockSpec((1, O), lambda b, n: (0, 0)),
            pl.BlockSpec((1, _RB, O), lambda b, n: (b, n, 0)),
        ],
        out_specs=pl.BlockSpec((1, O, _RB), lambda b, n: (b, 0, n)),
        out_shape=jax.ShapeDtypeStruct((B, O, N), jnp.float32),
    )(s1, s2, gamma.reshape(1, O), beta.reshape(1, O), y)
    return out


# speedup vs baseline: 3.4460x; 1.2581x over previous
import functools

import jax
import jax.numpy as jnp
from jax import lax
from jax.experimental import pallas as pl
from jax.experimental.pallas import tpu as pltpu
from jax.experimental.pallas import tpu_sc as plsc

_LEAKY = 0.01
_EPS = 1e-5
_BINS = 256
_GROUP = 128
_RB = 512
_CHUNK = 256
_CPR = _BINS // 128


def _logits(x_ref, nb):
    x_bf = x_ref[0].astype(jnp.bfloat16)
    xb = x_ref[0, :, pl.ds(nb * _RB, _RB)].astype(jnp.bfloat16)
    return lax.dot_general(
        xb, x_bf, dimension_numbers=(((0,), (0,)), ((), ())),
        preferred_element_type=jnp.float32)


def _quant_block(x_ref, qs_ref):
    nb = pl.program_id(0)
    N = x_ref.shape[2]
    logits = _logits(x_ref, nb)
    row_max = jnp.max(logits, axis=1, keepdims=True)
    row_min = jnp.min(logits, axis=1, keepdims=True)
    rng = row_max - row_min
    scale = jnp.where(rng > 0, _BINS / rng, 0.0)
    q = jnp.minimum(jnp.floor((logits - row_min) * scale), _BINS - 1)
    qp = q[:, :N // 2] * float(_BINS) + q[:, N // 2:]
    for g in range(_RB // _GROUP):
        qs_ref[pl.ds(g * (N // 2), N // 2), :] = (
            qp[g * _GROUP:(g + 1) * _GROUP, :].T)


def _sc_hist(qs_hbm, hist_hbm, buf0, buf1, hist, sem0, sem1):
    w = lax.axis_index("s") * 2 + lax.axis_index("c")
    n_groups = qs_hbm.shape[0] * 128 * 2 // (4096 * _GROUP)
    ngw = n_groups // 32
    qs_rows_per_group = 4096 * _GROUP // 128 // 2
    nchunk = qs_rows_per_group // _CHUNK
    bufs = (buf0, buf1)
    sems = (sem0, sem1)
    ones = jnp.ones((16,), jnp.float32)
    consts = [lax.iota(jnp.int32, 16) * _BINS + l * 16 * _BINS
              for l in range(8)]

    def start(buf, sem, base_row):
        pltpu.async_copy(
            qs_hbm.at[pl.ds(pl.multiple_of(base_row, _CHUNK), _CHUNK)],
            buf, sem)

    def wait(buf, sem):
        pltpu.make_async_copy(qs_hbm.at[pl.ds(0, _CHUNK)], buf, sem).wait()

    def process(buf):
        @plsc.parallel_loop(0, _CHUNK, unroll=2)
        def _(i):
            for l in range(8):
                v = buf[i, pl.ds(l * 16, 16)]
                vi = v.astype(jnp.int32)
                hi = lax.shift_right_logical(vi, _BINS.bit_length() - 1)
                lo = lax.bitwise_and(vi, _BINS - 1)
                plsc.addupdate_scatter(hist, [hi + consts[l]], ones)
                plsc.addupdate_scatter(hist, [lo + consts[l]], ones)

    for gi in range(ngw):
        g = w * ngw + gi

        @plsc.parallel_loop(0, _GROUP * _BINS // 16, unroll=4)
        def _(i):
            hist[pl.ds(i * 16, 16)] = jnp.zeros((16,), jnp.float32)

        base = g * qs_rows_per_group
        start(bufs[0], sems[0], base)

        def pair(j, _):
            c0 = base + 2 * j * _CHUNK
            wait(bufs[0], sems[0])
            start(bufs[1], sems[1], c0 + _CHUNK)
            process(bufs[0])
            wait(bufs[1], sems[1])

            @pl.when(j < nchunk // 2 - 1)
            def _():
                start(bufs[0], sems[0], c0 + 2 * _CHUNK)

            process(bufs[1])
            return 0
        lax.fori_loop(0, nchunk // 2, pair, 0)

        hb = pl.multiple_of(g * (_GROUP * _BINS), _GROUP * _BINS)
        pltpu.sync_copy(hist, hist_hbm.at[pl.ds(hb, _GROUP * _BINS)])


def _fused_block(x_ref, w_ref, h_ref, y_ref, sum_ref, ssq_ref, s_ref):
    nb = pl.program_id(0)
    N = x_ref.shape[2]
    k = int(round(N * 0.9))
    n_drop = N - k

    x_bf = x_ref[0].astype(jnp.bfloat16)

    @pl.when(nb == 0)
    def _():
        s_ref[...] = lax.dot_general(
            x_bf, w_ref[...].astype(jnp.bfloat16),
            dimension_numbers=(((0,), (0,)), ((), ())),
            preferred_element_type=jnp.float32,
        ).astype(jnp.bfloat16)

    logits = _logits(x_ref, nb)
    row_max = jnp.max(logits, axis=1, keepdims=True)
    row_min = jnp.min(logits, axis=1, keepdims=True)
    rng = row_max - row_min
    e = jnp.exp(logits - row_max)
    denom = jnp.sum(e, axis=1, keepdims=True)

    h = h_ref[...]
    c = h
    for sh in (1, 2, 4, 8, 16, 32, 64):
        z = jnp.zeros((c.shape[0], sh), jnp.float32)
        c = c + jnp.concatenate([z, c[:, :-sh]], axis=1)
    t = c[:, 127:128]
    rowsc = c.shape[0]
    pos = lax.broadcasted_iota(jnp.int32, (rowsc, 1), 0) % _CPR
    offs = jnp.zeros((rowsc, 1), jnp.float32)
    for sh in range(1, _CPR):
        z = jnp.zeros((sh, 1), jnp.float32)
        shifted = jnp.concatenate([z, t[:-sh, :]], axis=0)
        offs = offs + jnp.where(pos >= sh, shifted, 0.0)
    cum = c + offs
    ind = jnp.where(cum <= float(n_drop), 1.0, 0.0)
    s = jnp.sum(ind, axis=1, keepdims=True)
    bstarp1 = jnp.sum(s.reshape(-1, _CPR), axis=1, keepdims=True)
    thr = row_min + bstarp1 * (rng / _BINS)

    p = jnp.where(logits >= thr, e, 0.0) * (1.0 / denom)
    out = lax.dot_general(
        p.astype(jnp.bfloat16), s_ref[...],
        dimension_numbers=(((1,), (0,)), ((), ())),
        preferred_element_type=jnp.float32)
    z = jnp.where(out >= 0.0, out, _LEAKY * out)
    y_ref[0] = z
    sum_ref[0, 0] = jnp.sum(z, axis=0)
    ssq_ref[0, 0] = jnp.sum(z * z, axis=0)


def _bn_block(sum_ref, ssq_ref, g_ref, b_ref, y_ref, out_ref, *, count):
    tot = jnp.sum(sum_ref[:, 0, :], axis=0, keepdims=True)
    tot2 = jnp.sum(ssq_ref[:, 0, :], axis=0, keepdims=True)
    mean = tot / count
    var = tot2 / count - mean * mean
    inv = lax.rsqrt(var + _EPS)
    scale = inv * g_ref[...]
    shift = b_ref[...] - mean * scale
    z = y_ref[0] * scale + shift
    out_ref[0] = z.T


def kernel(input, W, gamma, beta):
    B, C, N = input.shape
    O = W.shape[1]
    NB = N // _RB
    mesh = plsc.VectorSubcoreMesh(core_axis_name="c", subcore_axis_name="s")

    quant = pl.pallas_call(
        _quant_block,
        grid=(NB,),
        in_specs=[pl.BlockSpec((1, C, N), lambda n: (0, 0, 0))],
        out_specs=pl.BlockSpec((_RB * N // 256, 128), lambda n: (n, 0)),
        out_shape=jax.ShapeDtypeStruct((N * N // 256, 128), jnp.float32),
    )

    sc_hist = pl.kernel(
        _sc_hist,
        mesh=mesh,
        compiler_params=pltpu.CompilerParams(needs_layout_passes=False),
        out_type=jax.ShapeDtypeStruct((N * _BINS,), jnp.float32),
        scratch_types=[
            pltpu.VMEM((_CHUNK, 128), jnp.float32),
            pltpu.VMEM((_CHUNK, 128), jnp.float32),
            pltpu.VMEM((_GROUP * _BINS,), jnp.float32),
            pltpu.SemaphoreType.DMA,
            pltpu.SemaphoreType.DMA,
        ],
    )

    fused = pl.pallas_call(
        _fused_block,
        grid=(NB,),
        in_specs=[
            pl.BlockSpec((1, C, N), lambda n: (0, 0, 0)),
            pl.BlockSpec((C, O), lambda n: (0, 0)),
            pl.BlockSpec((_RB * _BINS // 128, 128), lambda n: (n, 0)),
        ],
        out_specs=[
            pl.BlockSpec((1, _RB, O), lambda n: (0, n, 0)),
            pl.BlockSpec((1, 1, O), lambda n: (n, 0, 0)),
            pl.BlockSpec((1, 1, O), lambda n: (n, 0, 0)),
        ],
        out_shape=[
            jax.ShapeDtypeStruct((1, N, O), jnp.float32),
            jax.ShapeDtypeStruct((NB, 1, O), jnp.float32),
            jax.ShapeDtypeStruct((NB, 1, O), jnp.float32),
        ],
        scratch_shapes=[pltpu.VMEM((N, O), jnp.bfloat16)],
    )

    hists = []
    for b in range(B):
        qs_b = quant(input[b:b + 1])
        hists.append(sc_hist(qs_b).reshape(N * _BINS // 128, 128))
    ys, s1s, s2s = [], [], []
    for b in range(B):
        y_b, s1_b, s2_b = fused(input[b:b + 1], W, hists[b])
        ys.append(y_b)
        s1s.append(s1_b)
        s2s.append(s2_b)
    y = jnp.concatenate(ys, axis=0)
    s1 = jnp.concatenate(s1s, axis=0)
    s2 = jnp.concatenate(s2s, axis=0)

    out = pl.pallas_call(
        functools.partial(_bn_block, count=B * N),
        grid=(B, NB),
        in_specs=[
            pl.BlockSpec((B * NB, 1, O), lambda b, n: (0, 0, 0)),
            pl.BlockSpec((B * NB, 1, O), lambda b, n: (0, 0, 0)),
            pl.BlockSpec((1, O), lambda b, n: (0, 0)),
            pl.BlockSpec((1, O), lambda b, n: (0, 0)),
            pl.BlockSpec((1, _RB, O), lambda b, n: (b, n, 0)),
        ],
        out_specs=pl.BlockSpec((1, O, _RB), lambda b, n: (b, 0, n)),
        out_shape=jax.ShapeDtypeStruct((B, O, N), jnp.float32),
    )(s1, s2, gamma.reshape(1, O), beta.reshape(1, O), y)
    return out


# speedup vs baseline: 3.5598x; 1.0330x over previous
import functools

import jax
import jax.numpy as jnp
from jax import lax
from jax.experimental import pallas as pl
from jax.experimental.pallas import tpu as pltpu
from jax.experimental.pallas import tpu_sc as plsc

_LEAKY = 0.01
_EPS = 1e-5
_BINS = 256
_GROUP = 128
_RB = 512
_CHUNK = 256
_CPR = _BINS // 128


def _logits(x_ref, nb):
    x_bf = x_ref[0].astype(jnp.bfloat16)
    xb = x_ref[0, :, pl.ds(nb * _RB, _RB)].astype(jnp.bfloat16)
    return lax.dot_general(
        xb, x_bf, dimension_numbers=(((0,), (0,)), ((), ())),
        preferred_element_type=jnp.float32)


def _quant_block(x_ref, qs_ref, lo_ref, rng_ref):
    nb = pl.program_id(0)
    N = x_ref.shape[2]
    logits = _logits(x_ref, nb)
    row_max = jnp.max(logits, axis=1, keepdims=True)
    row_min = jnp.min(logits, axis=1, keepdims=True)
    rng = row_max - row_min
    scale = jnp.where(rng > 0, _BINS / rng, 0.0)
    q = jnp.minimum(jnp.floor((logits - row_min) * scale), _BINS - 1)
    qp = q[:, :N // 2] * float(_BINS) + q[:, N // 2:]
    for g in range(_RB // _GROUP):
        qs_ref[pl.ds(g * (N // 2), N // 2), :] = (
            qp[g * _GROUP:(g + 1) * _GROUP, :].T)
    lo_ref[...] = row_min
    rng_ref[...] = rng


def _sc_hist(qs_hbm, hist_hbm, buf0, buf1, hist, sem0, sem1):
    w = lax.axis_index("s") * 2 + lax.axis_index("c")
    n_groups = qs_hbm.shape[0] * 128 * 2 // (4096 * _GROUP)
    ngw = n_groups // 32
    qs_rows_per_group = 4096 * _GROUP // 128 // 2
    nchunk = qs_rows_per_group // _CHUNK
    bufs = (buf0, buf1)
    sems = (sem0, sem1)
    ones = jnp.ones((16,), jnp.float32)
    consts = [lax.iota(jnp.int32, 16) * _BINS + l * 16 * _BINS
              for l in range(8)]

    def start(buf, sem, base_row):
        pltpu.async_copy(
            qs_hbm.at[pl.ds(pl.multiple_of(base_row, _CHUNK), _CHUNK)],
            buf, sem)

    def wait(buf, sem):
        pltpu.make_async_copy(qs_hbm.at[pl.ds(0, _CHUNK)], buf, sem).wait()

    def process(buf):
        @plsc.parallel_loop(0, _CHUNK, unroll=2)
        def _(i):
            for l in range(8):
                v = buf[i, pl.ds(l * 16, 16)]
                vi = v.astype(jnp.int32)
                hi = lax.shift_right_logical(vi, _BINS.bit_length() - 1)
                lo = lax.bitwise_and(vi, _BINS - 1)
                plsc.addupdate_scatter(hist, [hi + consts[l]], ones)
                plsc.addupdate_scatter(hist, [lo + consts[l]], ones)

    for gi in range(ngw):
        g = w * ngw + gi

        @plsc.parallel_loop(0, _GROUP * _BINS // 16, unroll=4)
        def _(i):
            hist[pl.ds(i * 16, 16)] = jnp.zeros((16,), jnp.float32)

        base = g * qs_rows_per_group
        start(bufs[0], sems[0], base)

        def pair(j, _):
            c0 = base + 2 * j * _CHUNK
            wait(bufs[0], sems[0])
            start(bufs[1], sems[1], c0 + _CHUNK)
            process(bufs[0])
            wait(bufs[1], sems[1])

            @pl.when(j < nchunk // 2 - 1)
            def _():
                start(bufs[0], sems[0], c0 + 2 * _CHUNK)

            process(bufs[1])
            return 0
        lax.fori_loop(0, nchunk // 2, pair, 0)

        hb = pl.multiple_of(g * (_GROUP * _BINS), _GROUP * _BINS)
        pltpu.sync_copy(hist, hist_hbm.at[pl.ds(hb, _GROUP * _BINS)])


def _fused_block(x_ref, w_ref, h_ref, lo_ref, rng_ref, y_ref, sum_ref,
                 ssq_ref, s_ref):
    nb = pl.program_id(0)
    N = x_ref.shape[2]
    k = int(round(N * 0.9))
    n_drop = N - k

    x_bf = x_ref[0].astype(jnp.bfloat16)

    @pl.when(nb == 0)
    def _():
        s_ref[...] = lax.dot_general(
            x_bf, w_ref[...].astype(jnp.bfloat16),
            dimension_numbers=(((0,), (0,)), ((), ())),
            preferred_element_type=jnp.float32,
        ).astype(jnp.bfloat16)

    logits = _logits(x_ref, nb)
    row_min = lo_ref[...]
    rng = rng_ref[...]
    row_max = row_min + rng
    e = jnp.exp(logits - row_max)
    denom = jnp.sum(e, axis=1, keepdims=True)

    h = h_ref[...]
    c = h
    for sh in (1, 2, 4, 8, 16, 32, 64):
        z = jnp.zeros((c.shape[0], sh), jnp.float32)
        c = c + jnp.concatenate([z, c[:, :-sh]], axis=1)
    t = c[:, 127:128]
    rowsc = c.shape[0]
    pos = lax.broadcasted_iota(jnp.int32, (rowsc, 1), 0) % _CPR
    offs = jnp.zeros((rowsc, 1), jnp.float32)
    for sh in range(1, _CPR):
        z = jnp.zeros((sh, 1), jnp.float32)
        shifted = jnp.concatenate([z, t[:-sh, :]], axis=0)
        offs = offs + jnp.where(pos >= sh, shifted, 0.0)
    cum = c + offs
    ind = jnp.where(cum <= float(n_drop), 1.0, 0.0)
    s = jnp.sum(ind, axis=1, keepdims=True)
    bstarp1 = jnp.sum(s.reshape(-1, _CPR), axis=1, keepdims=True)
    thr = row_min + bstarp1 * (rng / _BINS)

    p = jnp.where(logits >= thr, e, 0.0) * (1.0 / denom)
    out = lax.dot_general(
        p.astype(jnp.bfloat16), s_ref[...],
        dimension_numbers=(((1,), (0,)), ((), ())),
        preferred_element_type=jnp.float32)
    z = jnp.where(out >= 0.0, out, _LEAKY * out)
    y_ref[0] = z
    sum_ref[0, 0] = jnp.sum(z, axis=0)
    ssq_ref[0, 0] = jnp.sum(z * z, axis=0)


def _bn_block(sum_ref, ssq_ref, g_ref, b_ref, y_ref, out_ref, *, count):
    tot = jnp.sum(sum_ref[:, 0, :], axis=0, keepdims=True)
    tot2 = jnp.sum(ssq_ref[:, 0, :], axis=0, keepdims=True)
    mean = tot / count
    var = tot2 / count - mean * mean
    inv = lax.rsqrt(var + _EPS)
    scale = inv * g_ref[...]
    shift = b_ref[...] - mean * scale
    z = y_ref[0] * scale + shift
    out_ref[0] = z.T


def kernel(input, W, gamma, beta):
    B, C, N = input.shape
    O = W.shape[1]
    NB = N // _RB
    mesh = plsc.VectorSubcoreMesh(core_axis_name="c", subcore_axis_name="s")

    quant = pl.pallas_call(
        _quant_block,
        grid=(NB,),
        in_specs=[pl.BlockSpec((1, C, N), lambda n: (0, 0, 0))],
        out_specs=[
            pl.BlockSpec((_RB * N // 256, 128), lambda n: (n, 0)),
            pl.BlockSpec((_RB, 1), lambda n: (n, 0)),
            pl.BlockSpec((_RB, 1), lambda n: (n, 0)),
        ],
        out_shape=[
            jax.ShapeDtypeStruct((N * N // 256, 128), jnp.float32),
            jax.ShapeDtypeStruct((N, 1), jnp.float32),
            jax.ShapeDtypeStruct((N, 1), jnp.float32),
        ],
    )

    sc_hist = pl.kernel(
        _sc_hist,
        mesh=mesh,
        compiler_params=pltpu.CompilerParams(needs_layout_passes=False),
        out_type=jax.ShapeDtypeStruct((N * _BINS,), jnp.float32),
        scratch_types=[
            pltpu.VMEM((_CHUNK, 128), jnp.float32),
            pltpu.VMEM((_CHUNK, 128), jnp.float32),
            pltpu.VMEM((_GROUP * _BINS,), jnp.float32),
            pltpu.SemaphoreType.DMA,
            pltpu.SemaphoreType.DMA,
        ],
    )

    fused = pl.pallas_call(
        _fused_block,
        grid=(NB,),
        in_specs=[
            pl.BlockSpec((1, C, N), lambda n: (0, 0, 0)),
            pl.BlockSpec((C, O), lambda n: (0, 0)),
            pl.BlockSpec((_RB * _BINS // 128, 128), lambda n: (n, 0)),
            pl.BlockSpec((_RB, 1), lambda n: (n, 0)),
            pl.BlockSpec((_RB, 1), lambda n: (n, 0)),
        ],
        out_specs=[
            pl.BlockSpec((1, _RB, O), lambda n: (0, n, 0)),
            pl.BlockSpec((1, 1, O), lambda n: (n, 0, 0)),
            pl.BlockSpec((1, 1, O), lambda n: (n, 0, 0)),
        ],
        out_shape=[
            jax.ShapeDtypeStruct((1, N, O), jnp.float32),
            jax.ShapeDtypeStruct((NB, 1, O), jnp.float32),
            jax.ShapeDtypeStruct((NB, 1, O), jnp.float32),
        ],
        scratch_shapes=[pltpu.VMEM((N, O), jnp.bfloat16)],
    )

    hists, los, rngs = [], [], []
    for b in range(B):
        qs_b, lo_b, rng_b = quant(input[b:b + 1])
        hists.append(sc_hist(qs_b).reshape(N * _BINS // 128, 128))
        los.append(lo_b)
        rngs.append(rng_b)
    ys, s1s, s2s = [], [], []
    for b in range(B):
        y_b, s1_b, s2_b = fused(input[b:b + 1], W, hists[b], los[b],
                                rngs[b])
        ys.append(y_b)
        s1s.append(s1_b)
        s2s.append(s2_b)
    y = jnp.concatenate(ys, axis=0)
    s1 = jnp.concatenate(s1s, axis=0)
    s2 = jnp.concatenate(s2s, axis=0)

    out = pl.pallas_call(
        functools.partial(_bn_block, count=B * N),
        grid=(B, NB),
        in_specs=[
            pl.BlockSpec((B * NB, 1, O), lambda b, n: (0, 0, 0)),
            pl.BlockSpec((B * NB, 1, O), lambda b, n: (0, 0, 0)),
            pl.BlockSpec((1, O), lambda b, n: (0, 0)),
            pl.BlockSpec((1, O), lambda b, n: (0, 0)),
            pl.BlockSpec((1, _RB, O), lambda b, n: (b, n, 0)),
        ],
        out_specs=pl.BlockSpec((1, O, _RB), lambda b, n: (b, 0, n)),
        out_shape=jax.ShapeDtypeStruct((B, O, N), jnp.float32),
    )(s1, s2, gamma.reshape(1, O), beta.reshape(1, O), y)
    return out
